# baseline (device time: 26490 ns/iter reference)
import jax
import jax.numpy as jnp
from jax import lax
from jax.experimental import pallas as pl
from jax.experimental.pallas import tpu as pltpu

TOKENS = 1024
DIM = 1024
VOCAB_PER_X = 8192
BLOCKS = 4
BLK = TOKENS // BLOCKS
CHUNKS = (32, 32, 64, 128)
OFFS = tuple(sum(CHUNKS[:i]) for i in range(len(CHUNKS)))
C = len(CHUNKS)
FWD_Z = (0, 1, 2)
FWD_Y = (3,)


def kernel(ids, E):
    my_x = lax.axis_index("x")
    my_y = lax.axis_index("y")
    my_z = lax.axis_index("z")

    blk = my_y * 2 + my_z
    ids_blk = lax.dynamic_slice(ids, (blk * BLK,), (BLK,))
    loc = ids_blk - my_x * VOCAB_PER_X
    mask = (loc >= 0) & (loc < VOCAB_PER_X)
    loc_c = jnp.where(mask, loc, 0).astype(jnp.int32)
    maskcol = mask.astype(jnp.bfloat16)[:, None]

    def body(loc_ref, mcol_ref, e_ref, out_ref,
             part32, partb, xrecv, gsems, send_sems, recv_sems):
        x = lax.axis_index("x")
        y = lax.axis_index("y")
        z = lax.axis_index("z")
        xn = (1 - x, y, z)
        yn = (x, 1 - y, z)
        zn = (x, y, 1 - z)

        b_own = (y * 2 + z) * BLK
        b_z = (y * 2 + (1 - z)) * BLK
        b_y = ((1 - y) * 2 + z) * BLK

        def rdma(src, dst, sem, dev):
            return pltpu.make_async_remote_copy(
                src_ref=src, dst_ref=dst,
                send_sem=send_sems.at[sem], recv_sem=recv_sems.at[sem],
                device_id=dev, device_id_type=pl.DeviceIdType.MESH,
            )

        def oslice(base, c):
            return out_ref.at[pl.ds(base + OFFS[c], CHUNKS[c]), :]

        def cslice(c):
            return pl.ds(OFFS[c], CHUNKS[c])

        bar = pltpu.get_barrier_semaphore()
        for nbr in (xn, yn, zn):
            pl.semaphore_signal(
                bar, inc=1, device_id=nbr,
                device_id_type=pl.DeviceIdType.MESH,
            )

        gcps = []
        for c in range(C):
            for t in range(CHUNKS[c]):
                i = OFFS[c] + t
                cp = pltpu.make_async_copy(
                    e_ref.at[loc_ref[i]], part32.at[i], gsems.at[c])
                gcps.append(cp)
                cp.start()

        r1 = []
        for c in range(C):
            for t in range(CHUNKS[c]):
                gcps[OFFS[c] + t].wait()
            sl = cslice(c)
            partb[sl, :] = jnp.where(
                mcol_ref[sl, :] != 0,
                part32[sl, :].astype(jnp.bfloat16),
                jnp.bfloat16(0),
            )
            if c == 0:
                pl.semaphore_wait(bar, 3)
            r1.append(rdma(partb.at[sl, :], xrecv.at[sl, :], c, xn))
            r1[c].start()

        rz = []
        ry = []
        for c in range(C):
            r1[c].wait_recv()
            out_ref[pl.ds(b_own + OFFS[c], CHUNKS[c]), :] = (
                partb[cslice(c), :] + xrecv[cslice(c), :]
            )
            rz.append(rdma(oslice(b_own, c), oslice(b_own, c), C + c, zn))
            ry.append(rdma(oslice(b_own, c), oslice(b_own, c), 2 * C + c, yn))
            rz[c].start()
            ry[c].start()

        fwd = []
        for c in range(C):
            rz[c].wait_recv()
            if c in FWD_Y:
                fwd.append(rdma(oslice(b_z, c), oslice(b_z, c),
                                3 * C + c, yn))
                fwd[-1].start()
            ry[c].wait_recv()
            if c in FWD_Z:
                fwd.append(rdma(oslice(b_y, c), oslice(b_y, c),
                                3 * C + c, zn))
                fwd[-1].start()

        for r in fwd:
            r.wait_recv()

        for r in r1 + rz + ry + fwd:
            r.wait_send()

    return pl.pallas_call(
        body,
        out_shape=jax.ShapeDtypeStruct((TOKENS, DIM), jnp.bfloat16),
        in_specs=[
            pl.BlockSpec(memory_space=pltpu.SMEM),
            pl.BlockSpec(memory_space=pltpu.VMEM),
            pl.BlockSpec(memory_space=pl.ANY),
        ],
        out_specs=pl.BlockSpec(memory_space=pltpu.VMEM),
        scratch_shapes=[
            pltpu.VMEM((BLK, DIM), jnp.float32),
            pltpu.VMEM((BLK, DIM), jnp.bfloat16),
            pltpu.VMEM((BLK, DIM), jnp.bfloat16),
            pltpu.SemaphoreType.DMA((C,)),
            pltpu.SemaphoreType.DMA((4 * C,)),
            pltpu.SemaphoreType.DMA((4 * C,)),
        ],
        compiler_params=pltpu.CompilerParams(collective_id=0),
    )(loc_c, maskcol, E)


# device time: 26316 ns/iter; 1.0066x vs baseline; 1.0066x over previous
import jax
import jax.numpy as jnp
from jax import lax
from jax.experimental import pallas as pl
from jax.experimental.pallas import tpu as pltpu

TOKENS = 1024
DIM = 1024
VOCAB_PER_X = 8192
BLOCKS = 4
BLK = TOKENS // BLOCKS
CHUNKS = (32, 64, 128, 32)
OFFS = tuple(sum(CHUNKS[:i]) for i in range(len(CHUNKS)))
C = len(CHUNKS)
FWD_Z = (0, 1, 3)
FWD_Y = (2,)


def kernel(ids, E):
    my_x = lax.axis_index("x")
    my_y = lax.axis_index("y")
    my_z = lax.axis_index("z")

    blk = my_y * 2 + my_z
    ids_blk = lax.dynamic_slice(ids, (blk * BLK,), (BLK,))
    loc = ids_blk - my_x * VOCAB_PER_X
    mask = (loc >= 0) & (loc < VOCAB_PER_X)
    loc_c = jnp.where(mask, loc, 0).astype(jnp.int32)
    maskcol = mask.astype(jnp.bfloat16)[:, None]

    def body(loc_ref, mcol_ref, e_ref, out_ref,
             part32, partb, xrecv, gsems, send_sems, recv_sems):
        x = lax.axis_index("x")
        y = lax.axis_index("y")
        z = lax.axis_index("z")
        xn = (1 - x, y, z)
        yn = (x, 1 - y, z)
        zn = (x, y, 1 - z)

        b_own = (y * 2 + z) * BLK
        b_z = (y * 2 + (1 - z)) * BLK
        b_y = ((1 - y) * 2 + z) * BLK

        def rdma(src, dst, sem, dev):
            return pltpu.make_async_remote_copy(
                src_ref=src, dst_ref=dst,
                send_sem=send_sems.at[sem], recv_sem=recv_sems.at[sem],
                device_id=dev, device_id_type=pl.DeviceIdType.MESH,
            )

        def oslice(base, c):
            return out_ref.at[pl.ds(base + OFFS[c], CHUNKS[c]), :]

        def cslice(c):
            return pl.ds(OFFS[c], CHUNKS[c])

        bar = pltpu.get_barrier_semaphore()
        for nbr in (xn, yn, zn):
            pl.semaphore_signal(
                bar, inc=1, device_id=nbr,
                device_id_type=pl.DeviceIdType.MESH,
            )

        gcps = []
        for c in range(C):
            for t in range(CHUNKS[c]):
                i = OFFS[c] + t
                cp = pltpu.make_async_copy(
                    e_ref.at[loc_ref[i]], part32.at[i], gsems.at[c])
                gcps.append(cp)
                cp.start()

        r1 = []
        for c in range(C):
            for t in range(CHUNKS[c]):
                gcps[OFFS[c] + t].wait()
            sl = cslice(c)
            partb[sl, :] = jnp.where(
                mcol_ref[sl, :] != 0,
                part32[sl, :].astype(jnp.bfloat16),
                jnp.bfloat16(0),
            )
            if c == 0:
                pl.semaphore_wait(bar, 3)
            r1.append(rdma(partb.at[sl, :], xrecv.at[sl, :], c, xn))
            r1[c].start()

        rz = []
        ry = []
        for c in range(C):
            r1[c].wait_recv()
            out_ref[pl.ds(b_own + OFFS[c], CHUNKS[c]), :] = (
                partb[cslice(c), :] + xrecv[cslice(c), :]
            )
            rz.append(rdma(oslice(b_own, c), oslice(b_own, c), C + c, zn))
            ry.append(rdma(oslice(b_own, c), oslice(b_own, c), 2 * C + c, yn))
            rz[c].start()
            ry[c].start()

        fwd = []
        for c in range(C):
            rz[c].wait_recv()
            if c in FWD_Y:
                fwd.append(rdma(oslice(b_z, c), oslice(b_z, c),
                                3 * C + c, yn))
                fwd[-1].start()
            ry[c].wait_recv()
            if c in FWD_Z:
                fwd.append(rdma(oslice(b_y, c), oslice(b_y, c),
                                3 * C + c, zn))
                fwd[-1].start()

        for r in fwd:
            r.wait_recv()

        for r in r1 + rz + ry + fwd:
            r.wait_send()

    return pl.pallas_call(
        body,
        out_shape=jax.ShapeDtypeStruct((TOKENS, DIM), jnp.bfloat16),
        in_specs=[
            pl.BlockSpec(memory_space=pltpu.SMEM),
            pl.BlockSpec(memory_space=pltpu.VMEM),
            pl.BlockSpec(memory_space=pl.ANY),
        ],
        out_specs=pl.BlockSpec(memory_space=pltpu.VMEM),
        scratch_shapes=[
            pltpu.VMEM((BLK, DIM), jnp.float32),
            pltpu.VMEM((BLK, DIM), jnp.bfloat16),
            pltpu.VMEM((BLK, DIM), jnp.bfloat16),
            pltpu.SemaphoreType.DMA((C,)),
            pltpu.SemaphoreType.DMA((4 * C,)),
            pltpu.SemaphoreType.DMA((4 * C,)),
        ],
        compiler_params=pltpu.CompilerParams(collective_id=0),
    )(loc_c, maskcol, E)


# device time: 23638 ns/iter; 1.1207x vs baseline; 1.1133x over previous
import jax
import jax.numpy as jnp
from jax import lax
from jax.experimental import pallas as pl
from jax.experimental.pallas import tpu as pltpu

TOKENS = 1024
DIM = 1024
VOCAB_PER_X = 8192
BLOCKS = 4
BLK = TOKENS // BLOCKS
CHUNKS = (64, 64, 64, 64)
OFFS = tuple(sum(CHUNKS[:i]) for i in range(len(CHUNKS)))
C = len(CHUNKS)
FWD_Z = (0, 1)
FWD_Y = (2, 3)


def kernel(ids, E):
    my_x = lax.axis_index("x")
    my_y = lax.axis_index("y")
    my_z = lax.axis_index("z")

    blk = my_y * 2 + my_z
    ids_blk = lax.dynamic_slice(ids, (blk * BLK,), (BLK,))
    loc = ids_blk - my_x * VOCAB_PER_X
    mask = (loc >= 0) & (loc < VOCAB_PER_X)
    loc_c = jnp.where(mask, loc, 0).astype(jnp.int32)
    maskcol = mask.astype(jnp.bfloat16)[:, None]

    def body(loc_ref, mcol_ref, e_ref, out_ref,
             part32, partb, xrecv, gsems, send_sems, recv_sems):
        x = lax.axis_index("x")
        y = lax.axis_index("y")
        z = lax.axis_index("z")
        xn = (1 - x, y, z)
        yn = (x, 1 - y, z)
        zn = (x, y, 1 - z)

        b_own = (y * 2 + z) * BLK
        b_z = (y * 2 + (1 - z)) * BLK
        b_y = ((1 - y) * 2 + z) * BLK

        def rdma(src, dst, sem, dev):
            return pltpu.make_async_remote_copy(
                src_ref=src, dst_ref=dst,
                send_sem=send_sems.at[sem], recv_sem=recv_sems.at[sem],
                device_id=dev, device_id_type=pl.DeviceIdType.MESH,
            )

        def oslice(base, c):
            return out_ref.at[pl.ds(base + OFFS[c], CHUNKS[c]), :]

        def cslice(c):
            return pl.ds(OFFS[c], CHUNKS[c])

        bar = pltpu.get_barrier_semaphore()
        for nbr in (xn, yn, zn):
            pl.semaphore_signal(
                bar, inc=1, device_id=nbr,
                device_id_type=pl.DeviceIdType.MESH,
            )

        gcps = []
        for c in range(C):
            for t in range(CHUNKS[c]):
                i = OFFS[c] + t
                cp = pltpu.make_async_copy(
                    e_ref.at[loc_ref[i]], part32.at[i], gsems.at[c])
                gcps.append(cp)
                cp.start()

        r1 = []
        for c in range(C):
            for t in range(CHUNKS[c]):
                gcps[OFFS[c] + t].wait()
            sl = cslice(c)
            partb[sl, :] = jnp.where(
                mcol_ref[sl, :] != 0,
                part32[sl, :].astype(jnp.bfloat16),
                jnp.bfloat16(0),
            )
            if c == 0:
                pl.semaphore_wait(bar, 3)
            r1.append(rdma(partb.at[sl, :], xrecv.at[sl, :], c, xn))
            r1[c].start()

        rz = []
        ry = []
        for c in range(C):
            r1[c].wait_recv()
            out_ref[pl.ds(b_own + OFFS[c], CHUNKS[c]), :] = (
                partb[cslice(c), :] + xrecv[cslice(c), :]
            )
            rz.append(rdma(oslice(b_own, c), oslice(b_own, c), C + c, zn))
            ry.append(rdma(oslice(b_own, c), oslice(b_own, c), 2 * C + c, yn))
            rz[c].start()
            ry[c].start()

        fwd = []
        for c in range(C):
            rz[c].wait_recv()
            if c in FWD_Y:
                fwd.append(rdma(oslice(b_z, c), oslice(b_z, c),
                                3 * C + c, yn))
                fwd[-1].start()
            ry[c].wait_recv()
            if c in FWD_Z:
                fwd.append(rdma(oslice(b_y, c), oslice(b_y, c),
                                3 * C + c, zn))
                fwd[-1].start()

        for r in fwd:
            r.wait_recv()

        for r in r1 + rz + ry + fwd:
            r.wait_send()

    return pl.pallas_call(
        body,
        out_shape=jax.ShapeDtypeStruct((TOKENS, DIM), jnp.bfloat16),
        in_specs=[
            pl.BlockSpec(memory_space=pltpu.SMEM),
            pl.BlockSpec(memory_space=pltpu.VMEM),
            pl.BlockSpec(memory_space=pl.ANY),
        ],
        out_specs=pl.BlockSpec(memory_space=pltpu.VMEM),
        scratch_shapes=[
            pltpu.VMEM((BLK, DIM), jnp.float32),
            pltpu.VMEM((BLK, DIM), jnp.bfloat16),
            pltpu.VMEM((BLK, DIM), jnp.bfloat16),
            pltpu.SemaphoreType.DMA((C,)),
            pltpu.SemaphoreType.DMA((4 * C,)),
            pltpu.SemaphoreType.DMA((4 * C,)),
        ],
        compiler_params=pltpu.CompilerParams(collective_id=0),
    )(loc_c, maskcol, E)


# device time: 22913 ns/iter; 1.1561x vs baseline; 1.0316x over previous
import jax
import jax.numpy as jnp
from jax import lax
from jax.experimental import pallas as pl
from jax.experimental.pallas import tpu as pltpu

TOKENS = 1024
DIM = 1024
VOCAB_PER_X = 8192
BLOCKS = 4
BLK = TOKENS // BLOCKS
CHUNKS = (32,) * 8
OFFS = tuple(sum(CHUNKS[:i]) for i in range(len(CHUNKS)))
C = len(CHUNKS)
FWD_Z = (0, 1, 2, 3)
FWD_Y = (4, 5, 6, 7)


def kernel(ids, E):
    my_x = lax.axis_index("x")
    my_y = lax.axis_index("y")
    my_z = lax.axis_index("z")

    blk = my_y * 2 + my_z
    ids_blk = lax.dynamic_slice(ids, (blk * BLK,), (BLK,))
    loc = ids_blk - my_x * VOCAB_PER_X
    mask = (loc >= 0) & (loc < VOCAB_PER_X)
    loc_c = jnp.where(mask, loc, 0).astype(jnp.int32)
    maskcol = mask.astype(jnp.bfloat16)[:, None]

    def body(loc_ref, mcol_ref, e_ref, out_ref,
             part32, partb, xrecv, gsems, send_sems, recv_sems):
        x = lax.axis_index("x")
        y = lax.axis_index("y")
        z = lax.axis_index("z")
        xn = (1 - x, y, z)
        yn = (x, 1 - y, z)
        zn = (x, y, 1 - z)

        b_own = (y * 2 + z) * BLK
        b_z = (y * 2 + (1 - z)) * BLK
        b_y = ((1 - y) * 2 + z) * BLK

        def rdma(src, dst, sem, dev):
            return pltpu.make_async_remote_copy(
                src_ref=src, dst_ref=dst,
                send_sem=send_sems.at[sem], recv_sem=recv_sems.at[sem],
                device_id=dev, device_id_type=pl.DeviceIdType.MESH,
            )

        def oslice(base, c):
            return out_ref.at[pl.ds(base + OFFS[c], CHUNKS[c]), :]

        def cslice(c):
            return pl.ds(OFFS[c], CHUNKS[c])

        bar = pltpu.get_barrier_semaphore()
        for nbr in (xn, yn, zn):
            pl.semaphore_signal(
                bar, inc=1, device_id=nbr,
                device_id_type=pl.DeviceIdType.MESH,
            )

        gcps = []
        for c in range(C):
            for t in range(CHUNKS[c]):
                i = OFFS[c] + t
                cp = pltpu.make_async_copy(
                    e_ref.at[loc_ref[i]], part32.at[i], gsems.at[c])
                gcps.append(cp)
                cp.start()

        r1 = []
        for c in range(C):
            for t in range(CHUNKS[c]):
                gcps[OFFS[c] + t].wait()
            sl = cslice(c)
            partb[sl, :] = jnp.where(
                mcol_ref[sl, :] != 0,
                part32[sl, :].astype(jnp.bfloat16),
                jnp.bfloat16(0),
            )
            if c == 0:
                pl.semaphore_wait(bar, 3)
            r1.append(rdma(partb.at[sl, :], xrecv.at[sl, :], c, xn))
            r1[c].start()

        rz = []
        ry = []
        for c in range(C):
            r1[c].wait_recv()
            out_ref[pl.ds(b_own + OFFS[c], CHUNKS[c]), :] = (
                partb[cslice(c), :] + xrecv[cslice(c), :]
            )
            rz.append(rdma(oslice(b_own, c), oslice(b_own, c), C + c, zn))
            ry.append(rdma(oslice(b_own, c), oslice(b_own, c), 2 * C + c, yn))
            rz[c].start()
            ry[c].start()

        fwd = []
        for c in range(C):
            rz[c].wait_recv()
            if c in FWD_Y:
                fwd.append(rdma(oslice(b_z, c), oslice(b_z, c),
                                3 * C + c, yn))
                fwd[-1].start()
            ry[c].wait_recv()
            if c in FWD_Z:
                fwd.append(rdma(oslice(b_y, c), oslice(b_y, c),
                                3 * C + c, zn))
                fwd[-1].start()

        for r in fwd:
            r.wait_recv()

        for r in r1 + rz + ry + fwd:
            r.wait_send()

    return pl.pallas_call(
        body,
        out_shape=jax.ShapeDtypeStruct((TOKENS, DIM), jnp.bfloat16),
        in_specs=[
            pl.BlockSpec(memory_space=pltpu.SMEM),
            pl.BlockSpec(memory_space=pltpu.VMEM),
            pl.BlockSpec(memory_space=pl.ANY),
        ],
        out_specs=pl.BlockSpec(memory_space=pltpu.VMEM),
        scratch_shapes=[
            pltpu.VMEM((BLK, DIM), jnp.float32),
            pltpu.VMEM((BLK, DIM), jnp.bfloat16),
            pltpu.VMEM((BLK, DIM), jnp.bfloat16),
            pltpu.SemaphoreType.DMA((C,)),
            pltpu.SemaphoreType.DMA((4 * C,)),
            pltpu.SemaphoreType.DMA((4 * C,)),
        ],
        compiler_params=pltpu.CompilerParams(collective_id=0),
    )(loc_c, maskcol, E)
